# baseline (speedup 1.0000x reference)
import functools

import jax
import jax.numpy as jnp
from jax import lax
from jax.experimental import pallas as pl
from jax.experimental.pallas import tpu as pltpu
from jax.experimental.pallas import tpu_sc as plsc

NC = 2
NS = 16
NW = NC * NS
B = 104
STRIP = 624
PAD = 16


def _sc_aggregate(node, src_pad, dst_pad, zrow, onesrow, *, n, d):
    e_pad = src_pad.shape[0]
    kpw = e_pad // (NW * B)
    epw = kpw * B
    rem = n - NS * STRIP
    nz = STRIP // B

    mesh = plsc.VectorSubcoreMesh(core_axis_name="c", subcore_axis_name="s")

    @functools.partial(
        pl.kernel,
        out_type=(
            jax.ShapeDtypeStruct((NC, n, d), jnp.float32),
            jax.ShapeDtypeStruct((NC, n, d), jnp.float32),
        ),
        mesh=mesh,
        scratch_types=[
            pltpu.VMEM((B,), jnp.int32),
            pltpu.VMEM((B,), jnp.int32),
            pltpu.VMEM((B,), jnp.int32),
            pltpu.VMEM((B,), jnp.int32),
            pltpu.VMEM((B, d), jnp.float32),
            pltpu.VMEM((B, d), jnp.float32),
            pltpu.VMEM((B, d), jnp.float32),
            pltpu.VMEM_SHARED((n + PAD, d), jnp.float32),
            pltpu.SemaphoreType.DMA,
            pltpu.SemaphoreType.DMA,
            pltpu.SemaphoreType.DMA,
            pltpu.SemaphoreType.DMA,
        ],
    )
    def agg(node_h, src_h, dst_h, zr_h, ones_h, sums_o, cnts_o,
            sidx0, sidx1, didx0, didx1, rows0, rows1, ones_v, accum,
            semg0, semg1, sems0, sems1):
        sidx = (sidx0, sidx1)
        didx = (didx0, didx1)
        rows = (rows0, rows1)
        semg = (semg0, semg1)
        sems = (sems0, sems1)
        cid = lax.axis_index("c")
        sid = lax.axis_index("s")
        wid = sid * NC + cid
        row0 = sid * STRIP
        ebase = wid * epw

        def zero_accum():
            for r in range(nz):
                pltpu.sync_copy(rows0, accum.at[pl.ds(row0 + r * B, B)])

            @pl.when(sid == 0)
            def _zero_rem():
                pltpu.sync_copy(rows0.at[pl.ds(0, rem + PAD)],
                                accum.at[pl.ds(NS * STRIP, rem + PAD)])

        def write_out(out_ref):
            for r in range(0, nz, 2):
                pltpu.sync_copy(accum.at[pl.ds(row0 + r * B, B)], rows0)
                pltpu.async_copy(rows0,
                                 out_ref.at[cid, pl.ds(row0 + r * B, B)],
                                 semg0)
                pltpu.sync_copy(accum.at[pl.ds(row0 + (r + 1) * B, B)],
                                rows1)
                pltpu.async_copy(rows1,
                                 out_ref.at[cid,
                                            pl.ds(row0 + (r + 1) * B, B)],
                                 semg1)
                pltpu.make_async_copy(
                    rows0, out_ref.at[cid, pl.ds(row0 + r * B, B)],
                    semg0).wait()
                pltpu.make_async_copy(
                    rows1, out_ref.at[cid, pl.ds(row0 + (r + 1) * B, B)],
                    semg1).wait()

            @pl.when(sid == 0)
            def _write_rem():
                pltpu.sync_copy(accum.at[pl.ds(NS * STRIP, rem)],
                                rows0.at[pl.ds(0, rem)])
                pltpu.sync_copy(rows0.at[pl.ds(0, rem)],
                                out_ref.at[cid, pl.ds(NS * STRIP, rem)])

        def idx_off(k):
            return pl.multiple_of(ebase + k * B, 8)

        pltpu.sync_copy(zr_h, rows0)
        pltpu.sync_copy(ones_h, ones_v)
        zero_accum()
        plsc.subcore_barrier()

        for b in range(2):
            pltpu.sync_copy(src_h.at[pl.ds(idx_off(b), B)], sidx[b])
            pltpu.sync_copy(dst_h.at[pl.ds(idx_off(b), B)], didx[b])
            pltpu.async_copy(node_h.at[sidx[b]], rows[b], semg[b])

        def step1(i, carry):
            for b in range(2):
                k = i * 2 + b
                pltpu.make_async_copy(node_h.at[sidx[b]], rows[b],
                                      semg[b]).wait()
                pltpu.async_copy(rows[b], accum.at[didx[b]], sems[b],
                                 add=True)
            for b in range(2):
                k = i * 2 + b
                pltpu.make_async_copy(rows[b], accum.at[didx[b]],
                                      sems[b]).wait()

                @pl.when(k + 2 < kpw)
                def _prefetch():
                    pltpu.sync_copy(src_h.at[pl.ds(idx_off(k + 2), B)],
                                    sidx[b])
                    pltpu.sync_copy(dst_h.at[pl.ds(idx_off(k + 2), B)],
                                    didx[b])
                    pltpu.async_copy(node_h.at[sidx[b]], rows[b], semg[b])
            return carry

        lax.fori_loop(0, kpw // 2, step1, 0)
        plsc.subcore_barrier()
        write_out(sums_o)
        plsc.subcore_barrier()

        pltpu.sync_copy(zr_h, rows0)
        zero_accum()
        plsc.subcore_barrier()

        for b in range(2):
            pltpu.async_copy(dst_h.at[pl.ds(idx_off(b), B)], didx[b],
                             semg[b])

        def step2(i, carry):
            for b in range(2):
                k = i * 2 + b
                pltpu.make_async_copy(dst_h.at[pl.ds(idx_off(k), B)],
                                      didx[b], semg[b]).wait()
                pltpu.async_copy(ones_v, accum.at[didx[b]], sems[b],
                                 add=True)
            for b in range(2):
                k = i * 2 + b
                pltpu.make_async_copy(ones_v, accum.at[didx[b]],
                                      sems[b]).wait()

                @pl.when(k + 2 < kpw)
                def _prefetch():
                    pltpu.async_copy(dst_h.at[pl.ds(idx_off(k + 2), B)],
                                     didx[b], semg[b])
            return carry

        lax.fori_loop(0, kpw // 2, step2, 0)
        plsc.subcore_barrier()
        write_out(cnts_o)

    return agg(node, src_pad, dst_pad, zrow, onesrow)


def _tc_finish(sums, cnts, x, wl, wr, *, n, d, bn):
    def body(s_ref, c_ref, x_ref, wl_ref, wr_ref, o_ref):
        s = s_ref[0] + s_ref[1]
        c = c_ref[0, :, 0:1] + c_ref[1, :, 0:1]
        mean = s / jnp.clip(c, 1.0, None)
        o_ref[...] = (
            jnp.dot(mean, wl_ref[...], preferred_element_type=jnp.float32)
            + jnp.dot(x_ref[...], wr_ref[...],
                      preferred_element_type=jnp.float32))

    grid = (n // bn,)
    return pl.pallas_call(
        body,
        grid=grid,
        in_specs=[
            pl.BlockSpec((NC, bn, d), lambda i: (0, i, 0)),
            pl.BlockSpec((NC, bn, d), lambda i: (0, i, 0)),
            pl.BlockSpec((bn, d), lambda i: (i, 0)),
            pl.BlockSpec((d, d), lambda i: (0, 0)),
            pl.BlockSpec((d, d), lambda i: (0, 0)),
        ],
        out_specs=pl.BlockSpec((bn, d), lambda i: (i, 0)),
        out_shape=jax.ShapeDtypeStruct((n, d), jnp.float32),
    )(sums, cnts, x, wl, wr)


def kernel(node_feature, edge_index, W_l, W_r):
    n, d = node_feature.shape
    e = edge_index.shape[1]
    e_pad = ((e + 2 * NW * B - 1) // (2 * NW * B)) * (2 * NW * B)
    npad = e_pad - e
    src_pad = jnp.concatenate(
        [edge_index[0], jnp.zeros((npad,), jnp.int32)])
    dst_pad = jnp.concatenate(
        [edge_index[1], jnp.full((npad,), n, jnp.int32)])
    zrow = jnp.zeros((B, d), jnp.float32)
    onesrow = jnp.ones((B, d), jnp.float32)
    sums, cnts = _sc_aggregate(node_feature, src_pad, dst_pad, zrow, onesrow,
                               n=n, d=d)
    return _tc_finish(sums, cnts, node_feature, W_l, W_r, n=n, d=d, bn=1000)


# speedup vs baseline: 1.4616x; 1.4616x over previous
import functools

import jax
import jax.numpy as jnp
from jax import lax
from jax.experimental import pallas as pl
from jax.experimental.pallas import tpu as pltpu
from jax.experimental.pallas import tpu_sc as plsc

NC = 2
NS = 16
NW = NC * NS
B = 104
STRIP = 624
PAD = 1024


def _sc_aggregate(node, src_pad, dst_pad, zrow, onesrow, *, n, d):
    e_pad = src_pad.shape[0]
    kpw = e_pad // (NW * B)
    epw = kpw * B
    rem = n - NS * STRIP
    nz = STRIP // B

    mesh = plsc.VectorSubcoreMesh(core_axis_name="c", subcore_axis_name="s")

    @functools.partial(
        pl.kernel,
        out_type=(
            jax.ShapeDtypeStruct((NC, n, d), jnp.float32),
            jax.ShapeDtypeStruct((NC, n, d), jnp.float32),
        ),
        mesh=mesh,
        scratch_types=[
            pltpu.VMEM((B,), jnp.int32),
            pltpu.VMEM((B,), jnp.int32),
            pltpu.VMEM((B, d), jnp.float32),
            pltpu.VMEM((B, d), jnp.float32),
            pltpu.VMEM_SHARED((n + PAD, d), jnp.float32),
            pltpu.SemaphoreType.DMA,
        ],
    )
    def agg(node_h, src_h, dst_h, zr_h, ones_h, sums_o, cnts_o,
            sidx_v, didx_v, rows_v, ones_v, accum, sem):
        cid = lax.axis_index("c")
        sid = lax.axis_index("s")
        wid = sid * NC + cid
        row0 = sid * STRIP
        ebase = wid * epw
        pad_tile = PAD // NS

        def zero_accum():
            for r in range(nz):
                pltpu.sync_copy(rows_v, accum.at[pl.ds(row0 + r * B, B)])
            pltpu.sync_copy(
                rows_v.at[pl.ds(0, pad_tile)],
                accum.at[pl.ds(NS * STRIP + rem + sid * pad_tile, pad_tile)])

            @pl.when(sid == 0)
            def _zero_rem():
                pltpu.sync_copy(rows_v.at[pl.ds(0, rem)],
                                accum.at[pl.ds(NS * STRIP, rem)])

        def write_out(out_ref):
            for r in range(nz):
                pltpu.sync_copy(accum.at[pl.ds(row0 + r * B, B)], rows_v)
                pltpu.sync_copy(rows_v,
                                out_ref.at[cid, pl.ds(row0 + r * B, B)])

            @pl.when(sid == 0)
            def _write_rem():
                pltpu.sync_copy(accum.at[pl.ds(NS * STRIP, rem)],
                                rows_v.at[pl.ds(0, rem)])
                pltpu.sync_copy(rows_v.at[pl.ds(0, rem)],
                                out_ref.at[cid, pl.ds(NS * STRIP, rem)])

        pltpu.sync_copy(zr_h, rows_v)
        pltpu.sync_copy(ones_h, ones_v)
        zero_accum()
        plsc.subcore_barrier()

        def step1(k, carry):
            off = pl.multiple_of(ebase + k * B, 8)
            pltpu.sync_copy(src_h.at[pl.ds(off, B)], sidx_v)
            pltpu.sync_copy(dst_h.at[pl.ds(off, B)], didx_v)
            pltpu.async_copy(node_h.at[sidx_v], rows_v, sem).wait()
            pltpu.sync_copy(rows_v, accum.at[didx_v], add=True)
            return carry

        lax.fori_loop(0, kpw, step1, 0)
        plsc.subcore_barrier()
        write_out(sums_o)
        plsc.subcore_barrier()

        pltpu.sync_copy(zr_h, rows_v)
        zero_accum()
        plsc.subcore_barrier()

        def step2(k, carry):
            off = pl.multiple_of(ebase + k * B, 8)
            pltpu.sync_copy(dst_h.at[pl.ds(off, B)], didx_v)
            pltpu.sync_copy(ones_v, accum.at[didx_v], add=True)
            return carry

        lax.fori_loop(0, kpw, step2, 0)
        plsc.subcore_barrier()
        write_out(cnts_o)

    return agg(node, src_pad, dst_pad, zrow, onesrow)


def _tc_finish(sums, cnts, x, wl, wr, *, n, d, bn):
    def body(s_ref, c_ref, x_ref, wl_ref, wr_ref, o_ref):
        s = s_ref[0] + s_ref[1]
        c = c_ref[0, :, 0:1] + c_ref[1, :, 0:1]
        mean = s / jnp.clip(c, 1.0, None)
        o_ref[...] = (
            jnp.dot(mean, wl_ref[...], preferred_element_type=jnp.float32)
            + jnp.dot(x_ref[...], wr_ref[...],
                      preferred_element_type=jnp.float32))

    grid = (n // bn,)
    return pl.pallas_call(
        body,
        grid=grid,
        in_specs=[
            pl.BlockSpec((NC, bn, d), lambda i: (0, i, 0)),
            pl.BlockSpec((NC, bn, d), lambda i: (0, i, 0)),
            pl.BlockSpec((bn, d), lambda i: (i, 0)),
            pl.BlockSpec((d, d), lambda i: (0, 0)),
            pl.BlockSpec((d, d), lambda i: (0, 0)),
        ],
        out_specs=pl.BlockSpec((bn, d), lambda i: (i, 0)),
        out_shape=jax.ShapeDtypeStruct((n, d), jnp.float32),
    )(sums, cnts, x, wl, wr)


def kernel(node_feature, edge_index, W_l, W_r):
    n, d = node_feature.shape
    e = edge_index.shape[1]
    e_pad = ((e + NW * B - 1) // (NW * B)) * (NW * B)
    npad = e_pad - e
    fill = jnp.arange(npad, dtype=jnp.int32)
    src_pad = jnp.concatenate([edge_index[0], fill % n])
    dst_pad = jnp.concatenate([edge_index[1], n + (fill % PAD)])
    zrow = jnp.zeros((B, d), jnp.float32)
    onesrow = jnp.ones((B, d), jnp.float32)
    sums, cnts = _sc_aggregate(node_feature, src_pad, dst_pad, zrow, onesrow,
                               n=n, d=d)
    return _tc_finish(sums, cnts, node_feature, W_l, W_r, n=n, d=d, bn=1000)


# speedup vs baseline: 2.0868x; 1.4278x over previous
import functools

import jax
import jax.numpy as jnp
from jax import lax
from jax.experimental import pallas as pl
from jax.experimental.pallas import tpu as pltpu
from jax.experimental.pallas import tpu_sc as plsc

NC = 2
NS = 16
NW = NC * NS
B = 104
STRIP = 624
PAD = 1024


def _sc_aggregate(node, src_pad, dst_pad, zrow, onesrow, *, n, d):
    e_pad = src_pad.shape[0]
    kpw = e_pad // (NW * B)
    epw = kpw * B
    rem = n - NS * STRIP
    nz = STRIP // B

    mesh = plsc.VectorSubcoreMesh(core_axis_name="c", subcore_axis_name="s")

    @functools.partial(
        pl.kernel,
        out_type=(
            jax.ShapeDtypeStruct((NC, n, d), jnp.float32),
            jax.ShapeDtypeStruct((NC, n, d), jnp.float32),
        ),
        mesh=mesh,
        scratch_types=[
            pltpu.VMEM((B,), jnp.int32),
            pltpu.VMEM((B,), jnp.int32),
            pltpu.VMEM((B,), jnp.int32),
            pltpu.VMEM((B,), jnp.int32),
            pltpu.VMEM((B, d), jnp.float32),
            pltpu.VMEM((B, d), jnp.float32),
            pltpu.VMEM((B, d), jnp.float32),
            pltpu.VMEM_SHARED((n + PAD, d), jnp.float32),
            pltpu.SemaphoreType.DMA,
            pltpu.SemaphoreType.DMA,
            pltpu.SemaphoreType.DMA,
            pltpu.SemaphoreType.DMA,
        ],
    )
    def agg(node_h, src_h, dst_h, zr_h, ones_h, sums_o, cnts_o,
            sidx0, sidx1, didx0, didx1, rows0, rows1, ones_v, accum,
            semg0, semg1, sems0, sems1):
        sidx = (sidx0, sidx1)
        didx = (didx0, didx1)
        rows = (rows0, rows1)
        semg = (semg0, semg1)
        sems = (sems0, sems1)
        rows_v = rows0
        cid = lax.axis_index("c")
        sid = lax.axis_index("s")
        wid = sid * NC + cid
        row0 = sid * STRIP
        ebase = wid * epw
        pad_tile = PAD // NS

        def zero_accum():
            for r in range(nz):
                pltpu.sync_copy(rows_v, accum.at[pl.ds(row0 + r * B, B)])
            pltpu.sync_copy(
                rows_v.at[pl.ds(0, pad_tile)],
                accum.at[pl.ds(NS * STRIP + rem + sid * pad_tile, pad_tile)])

            @pl.when(sid == 0)
            def _zero_rem():
                pltpu.sync_copy(rows_v.at[pl.ds(0, rem)],
                                accum.at[pl.ds(NS * STRIP, rem)])

        def write_out(out_ref):
            for r in range(nz):
                pltpu.sync_copy(accum.at[pl.ds(row0 + r * B, B)], rows_v)
                pltpu.sync_copy(rows_v,
                                out_ref.at[cid, pl.ds(row0 + r * B, B)])

            @pl.when(sid == 0)
            def _write_rem():
                pltpu.sync_copy(accum.at[pl.ds(NS * STRIP, rem)],
                                rows_v.at[pl.ds(0, rem)])
                pltpu.sync_copy(rows_v.at[pl.ds(0, rem)],
                                out_ref.at[cid, pl.ds(NS * STRIP, rem)])

        pltpu.sync_copy(zr_h, rows_v)
        pltpu.sync_copy(ones_h, ones_v)
        zero_accum()
        plsc.subcore_barrier()

        def idx_off(k):
            return pl.multiple_of(ebase + k * B, 8)

        for b in range(2):
            pltpu.sync_copy(src_h.at[pl.ds(idx_off(b), B)], sidx[b])
            pltpu.sync_copy(dst_h.at[pl.ds(idx_off(b), B)], didx[b])
            pltpu.async_copy(node_h.at[sidx[b]], rows[b], semg[b])

        def step1(i, carry):
            for b in range(2):
                pltpu.make_async_copy(node_h.at[sidx[b]], rows[b],
                                      semg[b]).wait()
                pltpu.async_copy(rows[b], accum.at[didx[b]], sems[b],
                                 add=True)
            for b in range(2):
                k = i * 2 + b
                pltpu.make_async_copy(rows[b], accum.at[didx[b]],
                                      sems[b]).wait()

                @pl.when(k + 2 < kpw)
                def _prefetch():
                    pltpu.sync_copy(src_h.at[pl.ds(idx_off(k + 2), B)],
                                    sidx[b])
                    pltpu.sync_copy(dst_h.at[pl.ds(idx_off(k + 2), B)],
                                    didx[b])
                    pltpu.async_copy(node_h.at[sidx[b]], rows[b], semg[b])
            return carry

        lax.fori_loop(0, kpw // 2, step1, 0)
        plsc.subcore_barrier()
        write_out(sums_o)
        plsc.subcore_barrier()

        pltpu.sync_copy(zr_h, rows_v)
        zero_accum()
        plsc.subcore_barrier()

        for b in range(2):
            pltpu.async_copy(dst_h.at[pl.ds(idx_off(b), B)], didx[b],
                             semg[b])

        def step2(i, carry):
            for b in range(2):
                k = i * 2 + b
                pltpu.make_async_copy(dst_h.at[pl.ds(idx_off(k), B)],
                                      didx[b], semg[b]).wait()
                pltpu.async_copy(ones_v, accum.at[didx[b]], sems[b],
                                 add=True)
            for b in range(2):
                k = i * 2 + b
                pltpu.make_async_copy(ones_v, accum.at[didx[b]],
                                      sems[b]).wait()

                @pl.when(k + 2 < kpw)
                def _prefetch():
                    pltpu.async_copy(dst_h.at[pl.ds(idx_off(k + 2), B)],
                                     didx[b], semg[b])
            return carry

        lax.fori_loop(0, kpw // 2, step2, 0)
        plsc.subcore_barrier()
        write_out(cnts_o)

    return agg(node, src_pad, dst_pad, zrow, onesrow)


def _tc_finish(sums, cnts, x, wl, wr, *, n, d, bn):
    def body(s_ref, c_ref, x_ref, wl_ref, wr_ref, o_ref):
        s = s_ref[0] + s_ref[1]
        c = c_ref[0, :, 0:1] + c_ref[1, :, 0:1]
        mean = s / jnp.clip(c, 1.0, None)
        o_ref[...] = (
            jnp.dot(mean, wl_ref[...], preferred_element_type=jnp.float32)
            + jnp.dot(x_ref[...], wr_ref[...],
                      preferred_element_type=jnp.float32))

    grid = (n // bn,)
    return pl.pallas_call(
        body,
        grid=grid,
        in_specs=[
            pl.BlockSpec((NC, bn, d), lambda i: (0, i, 0)),
            pl.BlockSpec((NC, bn, d), lambda i: (0, i, 0)),
            pl.BlockSpec((bn, d), lambda i: (i, 0)),
            pl.BlockSpec((d, d), lambda i: (0, 0)),
            pl.BlockSpec((d, d), lambda i: (0, 0)),
        ],
        out_specs=pl.BlockSpec((bn, d), lambda i: (i, 0)),
        out_shape=jax.ShapeDtypeStruct((n, d), jnp.float32),
    )(sums, cnts, x, wl, wr)


def kernel(node_feature, edge_index, W_l, W_r):
    n, d = node_feature.shape
    e = edge_index.shape[1]
    e_pad = ((e + 2 * NW * B - 1) // (2 * NW * B)) * (2 * NW * B)
    npad = e_pad - e
    fill = jnp.arange(npad, dtype=jnp.int32)
    src_pad = jnp.concatenate([edge_index[0], fill % n])
    dst_pad = jnp.concatenate([edge_index[1], n + (fill % PAD)])
    zrow = jnp.zeros((B, d), jnp.float32)
    onesrow = jnp.ones((B, d), jnp.float32)
    sums, cnts = _sc_aggregate(node_feature, src_pad, dst_pad, zrow, onesrow,
                               n=n, d=d)
    return _tc_finish(sums, cnts, node_feature, W_l, W_r, n=n, d=d, bn=1000)


# speedup vs baseline: 2.1178x; 1.0149x over previous
import functools

import jax
import jax.numpy as jnp
from jax import lax
from jax.experimental import pallas as pl
from jax.experimental.pallas import tpu as pltpu
from jax.experimental.pallas import tpu_sc as plsc

NC = 2
NS = 16
NW = NC * NS
B = 104
STRIP = 624
PAD = 1024


def _sc_aggregate(node, src_pad, dst_pad, zrow, onesrow, *, n, d):
    e_pad = src_pad.shape[0]
    kpw = e_pad // (NW * B)
    epw = kpw * B
    rem = n - NS * STRIP
    nz = STRIP // B

    mesh = plsc.VectorSubcoreMesh(core_axis_name="c", subcore_axis_name="s")

    @functools.partial(
        pl.kernel,
        out_type=(
            jax.ShapeDtypeStruct((NC, n, d), jnp.float32),
            jax.ShapeDtypeStruct((NC, n, d), jnp.float32),
        ),
        mesh=mesh,
        scratch_types=[
            pltpu.VMEM((B,), jnp.int32),
            pltpu.VMEM((B,), jnp.int32),
            pltpu.VMEM((B,), jnp.int32),
            pltpu.VMEM((B,), jnp.int32),
            pltpu.VMEM((B, d), jnp.float32),
            pltpu.VMEM((B, d), jnp.float32),
            pltpu.VMEM((B, d), jnp.float32),
            pltpu.VMEM_SHARED((n + PAD, d), jnp.float32),
            pltpu.SemaphoreType.DMA,
            pltpu.SemaphoreType.DMA,
            pltpu.SemaphoreType.DMA,
            pltpu.SemaphoreType.DMA,
        ],
    )
    def agg(node_h, src_h, dst_h, zr_h, ones_h, sums_o, cnts_o,
            sidx0, sidx1, didx0, didx1, rows0, rows1, ones_v, accum,
            semg0, semg1, sems0, sems1):
        sidx = (sidx0, sidx1)
        didx = (didx0, didx1)
        rows = (rows0, rows1)
        semg = (semg0, semg1)
        sems = (sems0, sems1)
        rows_v = rows0
        cid = lax.axis_index("c")
        sid = lax.axis_index("s")
        wid = sid * NC + cid
        row0 = sid * STRIP
        ebase = wid * epw
        pad_tile = PAD // NS

        def zero_accum():
            for r in range(nz):
                pltpu.sync_copy(rows_v, accum.at[pl.ds(row0 + r * B, B)])
            pltpu.sync_copy(
                rows_v.at[pl.ds(0, pad_tile)],
                accum.at[pl.ds(NS * STRIP + rem + sid * pad_tile, pad_tile)])

            @pl.when(sid == 0)
            def _zero_rem():
                pltpu.sync_copy(rows_v.at[pl.ds(0, rem)],
                                accum.at[pl.ds(NS * STRIP, rem)])

        def write_out(out_ref):
            for r in range(nz):
                b = r % 2
                sl = pl.ds(row0 + r * B, B)
                if r >= 2:
                    pltpu.make_async_copy(
                        rows[b], out_ref.at[cid, pl.ds(row0 + (r - 2) * B, B)],
                        sems[b]).wait()
                pltpu.sync_copy(accum.at[sl], rows[b])
                pltpu.async_copy(rows[b], out_ref.at[cid, sl], sems[b])
            for r in range(nz - 2, nz):
                b = r % 2
                pltpu.make_async_copy(
                    rows[b], out_ref.at[cid, pl.ds(row0 + r * B, B)],
                    sems[b]).wait()

            @pl.when(sid == 0)
            def _write_rem():
                pltpu.sync_copy(accum.at[pl.ds(NS * STRIP, rem)],
                                rows0.at[pl.ds(0, rem)])
                pltpu.sync_copy(rows0.at[pl.ds(0, rem)],
                                out_ref.at[cid, pl.ds(NS * STRIP, rem)])

        pltpu.sync_copy(zr_h, rows_v)
        pltpu.sync_copy(ones_h, ones_v)
        zero_accum()
        plsc.subcore_barrier()

        def idx_off(k):
            return pl.multiple_of(ebase + k * B, 8)

        for b in range(2):
            pltpu.sync_copy(src_h.at[pl.ds(idx_off(b), B)], sidx[b])
            pltpu.sync_copy(dst_h.at[pl.ds(idx_off(b), B)], didx[b])
            pltpu.async_copy(node_h.at[sidx[b]], rows[b], semg[b])

        def step1(i, carry):
            for b in range(2):
                pltpu.make_async_copy(node_h.at[sidx[b]], rows[b],
                                      semg[b]).wait()
                pltpu.async_copy(rows[b], accum.at[didx[b]], sems[b],
                                 add=True)
            for b in range(2):
                k = i * 2 + b
                pltpu.make_async_copy(rows[b], accum.at[didx[b]],
                                      sems[b]).wait()

                @pl.when(k + 2 < kpw)
                def _prefetch():
                    pltpu.sync_copy(src_h.at[pl.ds(idx_off(k + 2), B)],
                                    sidx[b])
                    pltpu.sync_copy(dst_h.at[pl.ds(idx_off(k + 2), B)],
                                    didx[b])
                    pltpu.async_copy(node_h.at[sidx[b]], rows[b], semg[b])
            return carry

        lax.fori_loop(0, kpw // 2, step1, 0)
        plsc.subcore_barrier()
        write_out(sums_o)
        plsc.subcore_barrier()

        pltpu.sync_copy(zr_h, rows_v)
        zero_accum()
        plsc.subcore_barrier()

        for b in range(2):
            pltpu.async_copy(dst_h.at[pl.ds(idx_off(b), B)], didx[b],
                             semg[b])

        def step2(i, carry):
            for b in range(2):
                k = i * 2 + b
                pltpu.make_async_copy(dst_h.at[pl.ds(idx_off(k), B)],
                                      didx[b], semg[b]).wait()
                pltpu.async_copy(ones_v, accum.at[didx[b]], sems[b],
                                 add=True)
            for b in range(2):
                k = i * 2 + b
                pltpu.make_async_copy(ones_v, accum.at[didx[b]],
                                      sems[b]).wait()

                @pl.when(k + 2 < kpw)
                def _prefetch():
                    pltpu.async_copy(dst_h.at[pl.ds(idx_off(k + 2), B)],
                                     didx[b], semg[b])
            return carry

        lax.fori_loop(0, kpw // 2, step2, 0)
        plsc.subcore_barrier()
        write_out(cnts_o)

    return agg(node, src_pad, dst_pad, zrow, onesrow)


def _tc_finish(sums, cnts, x, wl, wr, *, n, d, bn):
    def body(s_ref, c_ref, x_ref, wl_ref, wr_ref, o_ref):
        s = s_ref[0] + s_ref[1]
        c = c_ref[0, :, 0:1] + c_ref[1, :, 0:1]
        mean = s / jnp.clip(c, 1.0, None)
        o_ref[...] = (
            jnp.dot(mean, wl_ref[...], preferred_element_type=jnp.float32)
            + jnp.dot(x_ref[...], wr_ref[...],
                      preferred_element_type=jnp.float32))

    grid = (n // bn,)
    return pl.pallas_call(
        body,
        grid=grid,
        in_specs=[
            pl.BlockSpec((NC, bn, d), lambda i: (0, i, 0)),
            pl.BlockSpec((NC, bn, d), lambda i: (0, i, 0)),
            pl.BlockSpec((bn, d), lambda i: (i, 0)),
            pl.BlockSpec((d, d), lambda i: (0, 0)),
            pl.BlockSpec((d, d), lambda i: (0, 0)),
        ],
        out_specs=pl.BlockSpec((bn, d), lambda i: (i, 0)),
        out_shape=jax.ShapeDtypeStruct((n, d), jnp.float32),
    )(sums, cnts, x, wl, wr)


def kernel(node_feature, edge_index, W_l, W_r):
    n, d = node_feature.shape
    e = edge_index.shape[1]
    e_pad = ((e + 2 * NW * B - 1) // (2 * NW * B)) * (2 * NW * B)
    npad = e_pad - e
    fill = jnp.arange(npad, dtype=jnp.int32)
    src_pad = jnp.concatenate([edge_index[0], fill % n])
    dst_pad = jnp.concatenate([edge_index[1], n + (fill % PAD)])
    zrow = jnp.zeros((B, d), jnp.float32)
    onesrow = jnp.ones((B, d), jnp.float32)
    sums, cnts = _sc_aggregate(node_feature, src_pad, dst_pad, zrow, onesrow,
                               n=n, d=d)
    return _tc_finish(sums, cnts, node_feature, W_l, W_r, n=n, d=d, bn=1000)
